# baseline (speedup 1.0000x reference)
import functools

import jax
import jax.numpy as jnp
from jax import lax
from jax.experimental import pallas as pl
from jax.experimental.pallas import tpu as pltpu
from jax.experimental.pallas import tpu_sc as plsc

NUM_EMBEDDINGS = 1000000
EMBEDDING_DIM = 32
BATCH = 16384
N_FIELDS = 26

NUM_CORES = 2
NUM_SUBCORES = 16
NW = NUM_CORES * NUM_SUBCORES
L = 16

VCHUNK = 1000
N_VCHUNKS = NUM_EMBEDDINGS // VCHUNK

ROWS_PER_W = BATCH // NW
R_CHUNK = 32
N_CHUNKS = ROWS_PER_W // R_CHUNK
NBUF = 4


@functools.partial(
    pl.kernel,
    out_type=jax.ShapeDtypeStruct((NUM_EMBEDDINGS, EMBEDDING_DIM), jnp.float32),
    mesh=plsc.VectorSubcoreMesh(core_axis_name="c", subcore_axis_name="s"),
    scratch_types=[
        pltpu.VMEM((2, EMBEDDING_DIM, VCHUNK), jnp.float32),
        pltpu.VMEM((VCHUNK, EMBEDDING_DIM), jnp.float32),
        pltpu.SemaphoreType.DMA,
        pltpu.SemaphoreType.DMA,
    ],
    compiler_params=pltpu.CompilerParams(
        use_tc_tiling_on_sc=False, needs_layout_passes=False),
)
def _transpose_sc(wt_hbm, out_hbm, src_v, dst_v, sem_i, sem_o):
    wid = lax.axis_index("s") * NUM_CORES + lax.axis_index("c")
    lanes = lax.iota(jnp.int32, L)
    n_mine = (N_VCHUNKS - wid + NW - 1) // NW

    def issue_load(q):
        c0 = (wid + q * NW) * VCHUNK
        sb = lax.rem(q, 2)
        for e in range(EMBEDDING_DIM):
            pltpu.async_copy(
                wt_hbm.at[e, pl.ds(c0, VCHUNK)], src_v.at[sb, e], sem_i)

    def wait_load():
        for e in range(EMBEDDING_DIM):
            pltpu.make_async_copy(
                wt_hbm.at[0, pl.ds(0, VCHUNK)], src_v.at[0, 0], sem_i).wait()

    def wait_store():
        pltpu.make_async_copy(
            dst_v, out_hbm.at[pl.ds(0, VCHUNK)], sem_o).wait()

    issue_load(0)

    def body(q, _):
        sb = lax.rem(q, 2)
        wait_load()

        @pl.when(q + 1 < n_mine)
        def _():
            issue_load(q + 1)

        @pl.when(q >= 1)
        def _():
            wait_store()

        def tgroup(p, _):
            vvec = p * L + lanes
            for e in range(EMBEDDING_DIM):
                vals = plsc.load_gather(
                    src_v.at[sb], [jnp.full((L,), e, jnp.int32), vvec])
                plsc.store_scatter(
                    dst_v, [vvec, jnp.full((L,), e, jnp.int32)], vals)
            return 0

        lax.fori_loop(0, VCHUNK // L, tgroup, 0)

        tmask = lanes < (VCHUNK % L)
        tvec = jnp.minimum((VCHUNK // L) * L + lanes, VCHUNK - 1)
        for e in range(EMBEDDING_DIM):
            esplat = jnp.full((L,), e, jnp.int32)
            vals = plsc.load_gather(src_v.at[sb], [esplat, tvec], mask=tmask)
            plsc.store_scatter(dst_v, [tvec, esplat], vals, mask=tmask)
        pltpu.async_copy(
            dst_v, out_hbm.at[pl.ds((wid + q * NW) * VCHUNK, VCHUNK)], sem_o)
        return 0

    lax.fori_loop(0, n_mine, body, 0)
    wait_store()


@functools.partial(
    pl.kernel,
    out_type=jax.ShapeDtypeStruct((BATCH, N_FIELDS, EMBEDDING_DIM), jnp.float32),
    mesh=plsc.VectorSubcoreMesh(core_axis_name="c", subcore_axis_name="s"),
    scratch_types=[
        pltpu.VMEM((ROWS_PER_W, N_FIELDS), jnp.int32),
        pltpu.VMEM((NBUF, R_CHUNK, N_FIELDS, EMBEDDING_DIM), jnp.float32),
        pltpu.SemaphoreType.DMA,
        pltpu.SemaphoreType.DMA,
    ],
    compiler_params=pltpu.CompilerParams(use_tc_tiling_on_sc=False),
)
def _gather_sc(table_hbm, idx_hbm, out_hbm, idx_v, rows_v, sem_g, sem_s):
    wid = lax.axis_index("s") * NUM_CORES + lax.axis_index("c")
    base = wid * ROWS_PER_W

    pltpu.sync_copy(idx_hbm.at[pl.ds(base, ROWS_PER_W)], idx_v)

    def gather(i):
        b = i % NBUF

        def issue(j, _):
            pltpu.async_copy(
                table_hbm.at[idx_v.at[i * R_CHUNK + j]],
                rows_v.at[b, j], sem_g)
            return 0

        lax.fori_loop(0, R_CHUNK, issue, 0)
        return pltpu.make_async_copy(
            out_hbm.at[pl.ds(0, R_CHUNK)], rows_v.at[b], sem_g)

    def store(i):
        return pltpu.async_copy(
            rows_v.at[i % NBUF],
            out_hbm.at[pl.ds(base + i * R_CHUNK, R_CHUNK)], sem_s)

    gathers = [gather(0), gather(1)]
    stores = []
    for i in range(N_CHUNKS):
        gathers[i].wait()
        nxt = i + 2
        if nxt < N_CHUNKS:
            if nxt >= NBUF:
                stores[nxt - NBUF].wait()
            gathers.append(gather(nxt))
        stores.append(store(i))
    for j in range(max(0, N_CHUNKS - NBUF), N_CHUNKS):
        stores[j].wait()


def kernel(input, weight):
    table = _transpose_sc(weight.T)
    return _gather_sc(table, input)


# speedup vs baseline: 4.4170x; 4.4170x over previous
import functools

import jax
import jax.numpy as jnp
from jax import lax
from jax.experimental import pallas as pl
from jax.experimental.pallas import tpu as pltpu
from jax.experimental.pallas import tpu_sc as plsc

NUM_EMBEDDINGS = 1000000
EMBEDDING_DIM = 32
BATCH = 16384
N_FIELDS = 26

NUM_CORES = 2
NUM_SUBCORES = 16
NW = NUM_CORES * NUM_SUBCORES
ROWS_PER_W = BATCH // NW
R_CHUNK = 32
N_CHUNKS = ROWS_PER_W // R_CHUNK
NBUF = 4


@functools.partial(
    pl.kernel,
    out_type=jax.ShapeDtypeStruct((BATCH, N_FIELDS, EMBEDDING_DIM), jnp.float32),
    mesh=plsc.VectorSubcoreMesh(core_axis_name="c", subcore_axis_name="s"),
    scratch_types=[
        pltpu.VMEM((ROWS_PER_W, N_FIELDS), jnp.int32),
        pltpu.VMEM((NBUF, R_CHUNK, N_FIELDS, EMBEDDING_DIM), jnp.float32),
        pltpu.SemaphoreType.DMA,
        pltpu.SemaphoreType.DMA,
    ],
    compiler_params=pltpu.CompilerParams(use_tc_tiling_on_sc=False),
)
def _gather_sc(table_hbm, idx_hbm, out_hbm, idx_v, rows_v, sem_g, sem_s):
    wid = lax.axis_index("s") * NUM_CORES + lax.axis_index("c")
    base = wid * ROWS_PER_W

    pltpu.sync_copy(idx_hbm.at[pl.ds(base, ROWS_PER_W)], idx_v)

    def gather(i):
        b = i % NBUF

        def issue(j, _):
            pltpu.async_copy(
                table_hbm.at[idx_v.at[i * R_CHUNK + j]],
                rows_v.at[b, j], sem_g)
            return 0

        lax.fori_loop(0, R_CHUNK, issue, 0)
        return pltpu.make_async_copy(
            out_hbm.at[pl.ds(0, R_CHUNK)], rows_v.at[b], sem_g)

    def store(i):
        return pltpu.async_copy(
            rows_v.at[i % NBUF],
            out_hbm.at[pl.ds(base + i * R_CHUNK, R_CHUNK)], sem_s)

    gathers = [gather(0), gather(1)]
    stores = []
    for i in range(N_CHUNKS):
        gathers[i].wait()
        nxt = i + 2
        if nxt < N_CHUNKS:
            if nxt >= NBUF:
                stores[nxt - NBUF].wait()
            gathers.append(gather(nxt))
        stores.append(store(i))
    for j in range(max(0, N_CHUNKS - NBUF), N_CHUNKS):
        stores[j].wait()


def kernel(input, weight):
    return _gather_sc(weight, input)
